# baseline (speedup 1.0000x reference)
import functools

import jax
import jax.numpy as jnp
from jax import lax
from jax.experimental import pallas as pl
from jax.experimental.pallas import tpu as pltpu
from jax.experimental.pallas import tpu_sc as plsc

VS = 1000000
DS = 64
B = 16384
C = 20
N = 20

NC = 2
NS = 16
L = 16
NW = NC * NS
BPW = B // NW
CB = 8
NCHUNK = BPW // CB
SEG = 80
NSEG = CB * C // SEG

_LN2 = 0.6931471805599453
_LOG2_COEF = (
    -3.235854911107787, 7.086135972074948, -7.393883925453409,
    5.6658952659659345, -2.905906931388781, 0.9459083880958161,
    -0.17673384211718712, 0.014440352491874364,
)


def _neg_log(y):
    bits = lax.bitcast_convert_type(y, jnp.int32)
    e = ((bits >> 23) & 0xFF) - 127
    m = lax.bitcast_convert_type((bits & 0x7FFFFF) | 0x3F800000, jnp.float32)
    p = jnp.full((L,), _LOG2_COEF[7], jnp.float32)
    for k in range(6, -1, -1):
        p = p * m + _LOG2_COEF[k]
    return -(e.astype(jnp.float32) + p) * _LN2


def _sc_body(wrd_h, ctx_h, neg_h, cat_h, out_h,
             widx_v, cidx_v, nidx_v, wsh_v, csh_v, nsh_v,
             emb0_v, all0_v, emb1_v, all1_v, acc_v, sem0, sem1):
    cid = lax.axis_index("c")
    sid = lax.axis_index("s")
    wid = sid * NC + cid

    iota16 = lax.iota(jnp.int32, L)

    pltpu.sync_copy(wrd_h.at[pl.ds(wid * (BPW // L), BPW // L)], widx_v)
    pltpu.sync_copy(ctx_h.at[pl.ds(wid * (BPW * C // SEG), BPW * C // SEG)],
                    cidx_v)
    pltpu.sync_copy(neg_h.at[pl.ds(wid * (BPW * N // SEG), BPW * N // SEG)],
                    nidx_v)

    def split_w(r, c):
        raw = widx_v[r, pl.ds(0, L)]
        wsh_v[pl.ds(r * L, L)] = (raw & 1) << 4
        widx_v[r, pl.ds(0, L)] = raw >> 1
        return c
    lax.fori_loop(0, BPW // L, split_w, 0)

    def split_cn(r, c):
        for k in range(SEG // L):
            sl = pl.ds(k * L, L)
            craw = cidx_v[r, sl]
            csh_v[pl.ds(r * SEG + k * L, L)] = (craw & 1) << 4
            cidx_v[r, sl] = craw >> 1
            nraw = nidx_v[r, sl]
            nsh_v[pl.ds(r * SEG + k * L, L)] = (nraw & 1) << 4
            nidx_v[r, sl] = nraw >> 1
        return c
    lax.fori_loop(0, BPW * C // SEG, split_cn, 0)

    bufs = ((emb0_v, all0_v, sem0), (emb1_v, all1_v, sem1))

    def issue(irow, half, slot):
        emb_v, all_v, sem = bufs[slot]
        jc = 2 * irow + half
        pltpu.async_copy(
            cat_h.at[widx_v.at[irow, pl.ds(half * CB, CB)]], emb_v, sem)
        for s in range(NSEG):
            pltpu.async_copy(
                cat_h.at[cidx_v.at[jc * NSEG + s]],
                all_v.at[pl.ds(s * SEG, SEG)], sem)
            pltpu.async_copy(
                cat_h.at[nidx_v.at[jc * NSEG + s]],
                all_v.at[pl.ds(CB * C + s * SEG, SEG)], sem)

    def wait_slot(slot):
        emb_v, all_v, sem = bufs[slot]
        pltpu.make_async_copy(
            cat_h.at[widx_v.at[0, pl.ds(0, CB)]], emb_v, sem).wait()
        for s in range(NSEG):
            pltpu.make_async_copy(
                cat_h.at[cidx_v.at[s]],
                all_v.at[pl.ds(s * SEG, SEG)], sem).wait()
            pltpu.make_async_copy(
                cat_h.at[nidx_v.at[s]],
                all_v.at[pl.ds(CB * C + s * SEG, SEG)], sem).wait()

    xperms = [iota16 ^ (1 << j) for j in (3, 2, 1, 0)]
    merge_masks = [iota16 < 8, (iota16 & 4) == 0,
                   (iota16 & 2) == 0, (iota16 & 1) == 0]
    bitrev = (((iota16 & 1) << 3) | ((iota16 & 2) << 1)
              | ((iota16 & 4) >> 1) | ((iota16 & 8) >> 3))

    def unpack(w, sh):
        return lax.bitcast_convert_type((w >> sh) << 16, jnp.float32)

    def compute(irow, half, slot, acc_c):
        emb_v, all_v, _ = bufs[slot]
        jc = 2 * irow + half

        def bcast(vec, i):
            idx = jnp.full((L,), 0, jnp.int32) + i
            return vec.at[idx].get(mode="promise_in_bounds")

        def unit_body(u, acc_u):
            wvec = wsh_v[pl.ds(jc * CB, L)]
            e = []
            for h in range(2):
                eshv = bcast(wvec, 2 * u + h)
                e.append([unpack(emb_v[2 * u + h, pl.ds(k * L, L)], eshv)
                          for k in range(4)])
            s0 = u * (C + N)
            fb = jc * (CB * C) + s0
            cvecs = [csh_v[pl.ds(fb + g * L, L)] for g in range(3)]
            nvecs = [nsh_v[pl.ds(fb + g * L, L)] for g in range(3)]

            def dot_row(t):
                h, i = divmod(t, 2 * C)
                if i < C:
                    return h * C + i, True, h, True
                return h * C + (i - C), False, h, False

            for grp in range(5):
                accs = []
                for lane in range(L):
                    off, is_ctx, h, _ = dot_row(grp * L + lane)
                    row = (s0 + off) if is_ctx else (CB * C + s0 + off)
                    shv = bcast((cvecs if is_ctx else nvecs)[off // L],
                                off % L)
                    ek = e[h]
                    a = unpack(all_v[row, pl.ds(DS, L)], shv) * ek[0]
                    for k in range(1, 4):
                        a = a + unpack(
                            all_v[row, pl.ds(DS + k * L, L)], shv) * ek[k]
                    accs.append(a)
                for xp, msk in zip(xperms, merge_masks):
                    folded = [a + a.at[xp].get(mode="promise_in_bounds")
                              for a in accs]
                    accs = [jnp.where(msk, fa, fb)
                            for fa, fb in zip(folded[::2], folded[1::2])]
                x = accs[0]
                tvec = bitrev + (grp * L)
                pos = (tvec % 40) < C
                x = jnp.where(pos, x, -x)
                sg = 1.0 / (1.0 + jnp.exp(-x))
                y = jnp.clip(sg, 1e-6, 1.0 - 1e-6)
                acc_u = acc_u + _neg_log(y)
            return acc_u

        return lax.fori_loop(0, CB // 2, unit_body, acc_c)

    issue(jnp.int32(0), 0, 0)
    issue(jnp.int32(0), 1, 1)

    def pair_body(i, acc_c):
        nxt = jnp.minimum(i + 1, NCHUNK // 2 - 1)
        wait_slot(0)
        acc_c = compute(i, 0, 0, acc_c)
        issue(nxt, 0, 0)
        wait_slot(1)
        acc_c = compute(i, 1, 1, acc_c)
        issue(nxt, 1, 1)
        return acc_c

    acc = lax.fori_loop(0, NCHUNK // 2, pair_body,
                        jnp.zeros((L,), jnp.float32))
    wait_slot(0)
    wait_slot(1)
    acc_v[...] = acc
    pltpu.sync_copy(acc_v, out_h.at[wid])


_sc_call = functools.partial(
    pl.kernel,
    out_type=jax.ShapeDtypeStruct((NW, L), jnp.float32),
    mesh=plsc.VectorSubcoreMesh(
        core_axis_name="c", subcore_axis_name="s",
        num_cores=NC, num_subcores=NS),
    compiler_params=pltpu.CompilerParams(
        needs_layout_passes=False, use_tc_tiling_on_sc=False),
    scratch_types=[
        pltpu.VMEM((BPW // L, L), jnp.int32),
        pltpu.VMEM((BPW * C // SEG, SEG), jnp.int32),
        pltpu.VMEM((BPW * N // SEG, SEG), jnp.int32),
        pltpu.VMEM((BPW + L,), jnp.int32),
        pltpu.VMEM((BPW * C + L,), jnp.int32),
        pltpu.VMEM((BPW * N + L,), jnp.int32),
        pltpu.VMEM((CB, 2 * DS), jnp.int32),
        pltpu.VMEM((2 * CB * C, 2 * DS), jnp.int32),
        pltpu.VMEM((CB, 2 * DS), jnp.int32),
        pltpu.VMEM((2 * CB * C, 2 * DS), jnp.int32),
        pltpu.VMEM((L,), jnp.float32),
        pltpu.SemaphoreType.DMA,
        pltpu.SemaphoreType.DMA,
    ],
)(_sc_body)


VBLK = 8192


def _cat_body(it_ref, ot_ref, out_ref):
    ident = (lax.broadcasted_iota(jnp.int32, (DS, DS), 0)
             == lax.broadcasted_iota(jnp.int32, (DS, DS), 1)
             ).astype(jnp.float32)
    dn = (((0,), (0,)), ((), ()))

    def half(t_ref):
        t = lax.dot_general(t_ref[...], ident, dn,
                            preferred_element_type=jnp.float32)
        t16 = lax.bitcast_convert_type(t.astype(jnp.bfloat16), jnp.int16)
        t3 = t16.reshape(VBLK // 2, 2, DS)
        lo = t3[:, 0, :].astype(jnp.int32) & 0xFFFF
        hi = t3[:, 1, :].astype(jnp.int32)
        return lo | (hi << 16)

    out_ref[...] = jnp.concatenate([half(it_ref), half(ot_ref)], axis=1)


def _make_cat(iT, oT):
    return pl.pallas_call(
        _cat_body,
        grid=((VS + VBLK - 1) // VBLK,),
        in_specs=[
            pl.BlockSpec((DS, VBLK), lambda i: (0, i)),
            pl.BlockSpec((DS, VBLK), lambda i: (0, i)),
        ],
        out_specs=pl.BlockSpec((VBLK // 2, 2 * DS), lambda i: (i, 0)),
        out_shape=jax.ShapeDtypeStruct((VS // 2, 2 * DS), jnp.int32),
    )(iT, oT)


def kernel(iEmb, oEmb, wrd, ctx, neg):
    wrd_i = wrd.astype(jnp.int32).reshape(B // L, L)
    ctx_i = ctx.astype(jnp.int32).reshape(B * C // SEG, SEG)
    neg_i = neg.astype(jnp.int32).reshape(B * N // SEG, SEG)
    cat = _make_cat(iEmb.T, oEmb.T)
    parts = _sc_call(wrd_i, ctx_i, neg_i, cat)
    return parts.sum() / (B * C)


# speedup vs baseline: 2.2793x; 2.2793x over previous
import functools

import jax
import jax.numpy as jnp
from jax import lax
from jax.experimental import pallas as pl
from jax.experimental.pallas import tpu as pltpu
from jax.experimental.pallas import tpu_sc as plsc

VS = 1000000
DS = 64
B = 16384
C = 20
N = 20

NC = 2
NS = 16
L = 16
NW = NC * NS
BPW = B // NW
CB = 8
NCHUNK = BPW // CB
SEG = 80
NSEG = CB * C // SEG

_LN2 = 0.6931471805599453
_LOG2_COEF = (
    -3.235854911107787, 7.086135972074948, -7.393883925453409,
    5.6658952659659345, -2.905906931388781, 0.9459083880958161,
    -0.17673384211718712, 0.014440352491874364,
)


def _neg_log(y):
    bits = lax.bitcast_convert_type(y, jnp.int32)
    e = ((bits >> 23) & 0xFF) - 127
    m = lax.bitcast_convert_type((bits & 0x7FFFFF) | 0x3F800000, jnp.float32)
    p = jnp.full((L,), _LOG2_COEF[7], jnp.float32)
    for k in range(6, -1, -1):
        p = p * m + _LOG2_COEF[k]
    return -(e.astype(jnp.float32) + p) * _LN2


def _sc_body(wrd_h, ctx_h, neg_h, cat_h, out_h,
             widx_v, cidx_v, nidx_v, emb0_v, all0_v, emb1_v, all1_v,
             acc_v, sem0, sem1):
    cid = lax.axis_index("c")
    sid = lax.axis_index("s")
    wid = sid * NC + cid

    iota16 = lax.iota(jnp.int32, L)

    pltpu.sync_copy(wrd_h.at[pl.ds(wid * (BPW // CB), BPW // CB)], widx_v)
    pltpu.sync_copy(ctx_h.at[pl.ds(wid * (BPW * C // SEG), BPW * C // SEG)],
                    cidx_v)
    pltpu.sync_copy(neg_h.at[pl.ds(wid * (BPW * N // SEG), BPW * N // SEG)],
                    nidx_v)

    bufs = ((emb0_v, all0_v, sem0), (emb1_v, all1_v, sem1))

    def issue(jc, slot):
        emb_v, all_v, sem = bufs[slot]
        pltpu.async_copy(cat_h.at[widx_v.at[jc]], emb_v, sem)
        for s in range(NSEG):
            pltpu.async_copy(
                cat_h.at[cidx_v.at[jc * NSEG + s]],
                all_v.at[pl.ds(s * SEG, SEG)], sem)
            pltpu.async_copy(
                cat_h.at[nidx_v.at[jc * NSEG + s]],
                all_v.at[pl.ds(CB * C + s * SEG, SEG)], sem)

    def wait_slot(slot):
        emb_v, all_v, sem = bufs[slot]
        pltpu.make_async_copy(cat_h.at[widx_v.at[0]], emb_v, sem).wait()
        for s in range(NSEG):
            pltpu.make_async_copy(
                cat_h.at[cidx_v.at[s]],
                all_v.at[pl.ds(s * SEG, SEG)], sem).wait()
            pltpu.make_async_copy(
                cat_h.at[nidx_v.at[s]],
                all_v.at[pl.ds(CB * C + s * SEG, SEG)], sem).wait()

    xperms = [iota16 ^ (1 << j) for j in (3, 2, 1, 0)]
    merge_masks = [iota16 < 8, (iota16 & 4) == 0,
                   (iota16 & 2) == 0, (iota16 & 1) == 0]
    bitrev = (((iota16 & 1) << 3) | ((iota16 & 2) << 1)
              | ((iota16 & 4) >> 1) | ((iota16 & 8) >> 3))

    def compute(slot, acc_c):
        emb_v, all_v, _ = bufs[slot]

        def unit_body(u, acc_u):
            e = [[emb_v[2 * u + h, pl.ds(k * L, L)] for k in range(4)]
                 for h in range(2)]
            s0 = u * (C + N)

            def dot_row(t):
                h, i = divmod(t, 2 * C)
                if i < C:
                    return s0 + h * C + i, h, True
                return CB * C + s0 + h * C + (i - C), h, False

            for grp in range(5):
                accs = []
                for lane in range(L):
                    row, h, _ = dot_row(grp * L + lane)
                    ek = e[h]
                    a = all_v[row, pl.ds(DS, L)] * ek[0]
                    for k in range(1, 4):
                        a = a + all_v[row, pl.ds(DS + k * L, L)] * ek[k]
                    accs.append(a)
                for xp, msk in zip(xperms, merge_masks):
                    folded = [a + a.at[xp].get(mode="promise_in_bounds")
                              for a in accs]
                    accs = [jnp.where(msk, fa, fb)
                            for fa, fb in zip(folded[::2], folded[1::2])]
                x = accs[0]
                tvec = bitrev + (grp * L)
                pos = (tvec % 40) < C
                x = jnp.where(pos, x, -x)
                sg = 1.0 / (1.0 + jnp.exp(-x))
                y = jnp.clip(sg, 1e-6, 1.0 - 1e-6)
                acc_u = acc_u + _neg_log(y)
            return acc_u

        return lax.fori_loop(0, CB // 2, unit_body, acc_c)

    issue(jnp.int32(0), 0)
    issue(jnp.int32(1), 1)

    def pair_body(i, acc_c):
        wait_slot(0)
        acc_c = compute(0, acc_c)
        issue(jnp.minimum(2 * i + 2, NCHUNK - 1), 0)
        wait_slot(1)
        acc_c = compute(1, acc_c)
        issue(jnp.minimum(2 * i + 3, NCHUNK - 1), 1)
        return acc_c

    acc = lax.fori_loop(0, NCHUNK // 2, pair_body,
                        jnp.zeros((L,), jnp.float32))
    wait_slot(0)
    wait_slot(1)
    acc_v[...] = acc
    pltpu.sync_copy(acc_v, out_h.at[wid])


_sc_call = functools.partial(
    pl.kernel,
    out_type=jax.ShapeDtypeStruct((NW, L), jnp.float32),
    mesh=plsc.VectorSubcoreMesh(
        core_axis_name="c", subcore_axis_name="s",
        num_cores=NC, num_subcores=NS),
    compiler_params=pltpu.CompilerParams(
        needs_layout_passes=False, use_tc_tiling_on_sc=False),
    scratch_types=[
        pltpu.VMEM((BPW // CB, CB), jnp.int32),
        pltpu.VMEM((BPW * C // SEG, SEG), jnp.int32),
        pltpu.VMEM((BPW * N // SEG, SEG), jnp.int32),
        pltpu.VMEM((CB, 2 * DS), jnp.float32),
        pltpu.VMEM((2 * CB * C, 2 * DS), jnp.float32),
        pltpu.VMEM((CB, 2 * DS), jnp.float32),
        pltpu.VMEM((2 * CB * C, 2 * DS), jnp.float32),
        pltpu.VMEM((L,), jnp.float32),
        pltpu.SemaphoreType.DMA,
        pltpu.SemaphoreType.DMA,
    ],
)(_sc_body)


VBLK = 16384


def _cat_body(it_ref, ot_ref, out_ref):
    ident = (lax.broadcasted_iota(jnp.int32, (DS, DS), 0)
             == lax.broadcasted_iota(jnp.int32, (DS, DS), 1)
             ).astype(jnp.float32)
    dn = (((0,), (0,)), ((), ()))
    out_ref[...] = jnp.concatenate(
        [lax.dot_general(it_ref[...], ident, dn,
                         preferred_element_type=jnp.float32),
         lax.dot_general(ot_ref[...], ident, dn,
                         preferred_element_type=jnp.float32)], axis=1)


def _make_cat(iT, oT):
    return pl.pallas_call(
        _cat_body,
        grid=((VS + VBLK - 1) // VBLK,),
        in_specs=[
            pl.BlockSpec((DS, VBLK), lambda i: (0, i)),
            pl.BlockSpec((DS, VBLK), lambda i: (0, i)),
        ],
        out_specs=pl.BlockSpec((VBLK, 2 * DS), lambda i: (i, 0)),
        out_shape=jax.ShapeDtypeStruct((VS, 2 * DS), jnp.float32),
    )(iT, oT)


def kernel(iEmb, oEmb, wrd, ctx, neg):
    wrd_i = wrd.astype(jnp.int32).reshape(B // CB, CB)
    ctx_i = ctx.astype(jnp.int32).reshape(B * C // SEG, SEG)
    neg_i = neg.astype(jnp.int32).reshape(B * N // SEG, SEG)
    cat = _make_cat(iEmb.T, oEmb.T)
    parts = _sc_call(wrd_i, ctx_i, neg_i, cat)
    return parts.sum() / (B * C)
